# baseline (speedup 1.0000x reference)
import functools

import jax
from jax import lax
import jax.numpy as jnp
from jax.experimental import pallas as pl
from jax.experimental.pallas import tpu as pltpu
from jax.experimental.pallas import tpu_sc as plsc

S = 2048
D = 1024
F = 2048
E = 8
K = 2
A = S * K
BM = 256
NB = A // BM
V = NB + E - 1

NC = 2
NS = 16
NW = NC * NS
TW = S // NW

def _sc_mesh():
    return plsc.VectorSubcoreMesh(core_axis_name="c", subcore_axis_name="s")


def _dispatch_body(x_hbm, idx_hbm, xs_hbm, ie_v, io_v, rows_v, sem):
    wid = lax.axis_index("s") * NC + lax.axis_index("c")
    pltpu.sync_copy(idx_hbm.at[wid, 0], ie_v)
    pltpu.sync_copy(idx_hbm.at[wid, 1], io_v)
    pltpu.sync_copy(x_hbm.at[pl.ds(wid * TW, TW)], rows_v)
    pltpu.async_copy(rows_v, xs_hbm.at[ie_v], sem).wait()
    pltpu.async_copy(rows_v, xs_hbm.at[io_v], sem).wait()


def _sc_dispatch(inputs, posT):
    kern = pl.kernel(
        _dispatch_body,
        mesh=_sc_mesh(),
        out_type=jax.ShapeDtypeStruct((A, D), jnp.float32),
        scratch_types=[
            pltpu.VMEM((TW,), jnp.int32),
            pltpu.VMEM((TW,), jnp.int32),
            pltpu.VMEM((TW, D), jnp.float32),
            pltpu.SemaphoreType.DMA,
        ],
    )
    return kern(inputs, posT)


def _uncombine_body(y_hbm, idx_hbm, ya_hbm, yb_hbm, ie_v, io_v, rows_v, sem):
    wid = lax.axis_index("s") * NC + lax.axis_index("c")
    pltpu.sync_copy(idx_hbm.at[wid, 0], ie_v)
    pltpu.sync_copy(idx_hbm.at[wid, 1], io_v)
    pltpu.async_copy(y_hbm.at[ie_v], rows_v, sem).wait()
    pltpu.sync_copy(rows_v, ya_hbm.at[pl.ds(wid * TW, TW)])
    pltpu.async_copy(y_hbm.at[io_v], rows_v, sem).wait()
    pltpu.sync_copy(rows_v, yb_hbm.at[pl.ds(wid * TW, TW)])


def _sc_uncombine(y, posT):
    kern = pl.kernel(
        _uncombine_body,
        mesh=_sc_mesh(),
        out_type=[
            jax.ShapeDtypeStruct((S, D), jnp.float32),
            jax.ShapeDtypeStruct((S, D), jnp.float32),
        ],
        scratch_types=[
            pltpu.VMEM((TW,), jnp.int32),
            pltpu.VMEM((TW,), jnp.int32),
            pltpu.VMEM((TW, D), jnp.float32),
            pltpu.SemaphoreType.DMA,
        ],
    )
    return kern(y, posT)


def _router_body(x_ref, wg_ref, gl_ref, laux_ref, idx_ref, wts_ref):
    x = x_ref[...]
    g = jnp.dot(x, wg_ref[...], preferred_element_type=jnp.float32)
    gl_ref[...] = g
    m = jnp.max(g, axis=1, keepdims=True)
    eg = jnp.exp(g - m)
    gates = eg / jnp.sum(eg, axis=1, keepdims=True)
    iota = jax.lax.broadcasted_iota(jnp.int32, (S, E), 1)
    i1 = jnp.min(jnp.where(g == m, iota, E), axis=1)
    sel1 = iota == i1[:, None]
    g2 = jnp.where(sel1, -jnp.inf, g)
    m2 = jnp.max(g2, axis=1, keepdims=True)
    i2 = jnp.min(jnp.where(g2 == m2, iota, E), axis=1)
    w1s = jax.nn.sigmoid(m[:, 0] - m2[:, 0])
    w2s = jax.nn.sigmoid(m2[:, 0] - m[:, 0])
    idx_ref[...] = jnp.concatenate([i1[:, None], i2[:, None]], axis=1)
    wts_ref[...] = jnp.concatenate([w1s[:, None], w2s[:, None]], axis=1)
    me = jnp.mean(gates, axis=0, keepdims=True)
    ce = jnp.mean(sel1.astype(jnp.float32), axis=0, keepdims=True)
    laux_ref[...] = jnp.sum(me * ce, axis=1, keepdims=True) * E


def _router(inputs, Wg):
    return pl.pallas_call(
        _router_body,
        out_shape=[
            jax.ShapeDtypeStruct((S, E), jnp.float32),
            jax.ShapeDtypeStruct((1, 1), jnp.float32),
            jax.ShapeDtypeStruct((S, K), jnp.int32),
            jax.ShapeDtypeStruct((S, K), jnp.float32),
        ],
    )(inputs, Wg)


def _expert_body(meta_ref, x_ref, w1_ref, w2_ref, y_ref):
    v = pl.program_id(0)
    lo = meta_ref[2 * V + v]
    hi = meta_ref[3 * V + v]

    @pl.when(hi > lo)
    def _():
        x = x_ref[...]
        h = jnp.dot(x, w1_ref[0], preferred_element_type=jnp.float32)
        h = h * jax.nn.sigmoid(h)
        y = jnp.dot(h, w2_ref[0], preferred_element_type=jnp.float32)
        r = (meta_ref[V + v] * BM
             + lax.broadcasted_iota(jnp.int32, (BM, 1), 0))
        mask = (r >= lo) & (r < hi)
        y_ref[...] = jnp.where(mask, y, y_ref[...])


def _expert_mm(meta, x_sorted, w1, w2):
    grid_spec = pltpu.PrefetchScalarGridSpec(
        num_scalar_prefetch=1,
        grid=(V,),
        in_specs=[
            pl.BlockSpec((BM, D), lambda v, m: (m[V + v], 0)),
            pl.BlockSpec((1, D, F), lambda v, m: (m[v], 0, 0)),
            pl.BlockSpec((1, F, D), lambda v, m: (m[v], 0, 0)),
        ],
        out_specs=pl.BlockSpec((BM, D), lambda v, m: (m[V + v], 0)),
    )
    return pl.pallas_call(
        _expert_body,
        grid_spec=grid_spec,
        out_shape=jax.ShapeDtypeStruct((A, D), jnp.float32),
        compiler_params=pltpu.CompilerParams(
            dimension_semantics=("arbitrary",)),
    )(meta, x_sorted, w1, w2)


def _combine_body(a_ref, b_ref, w_ref, o_ref):
    w = w_ref[...]
    o_ref[...] = a_ref[...] * w[:, 0:1] + b_ref[...] * w[:, 1:2]


def _combine(a, b, wts):
    nblk = S // BM
    return pl.pallas_call(
        _combine_body,
        grid=(nblk,),
        in_specs=[
            pl.BlockSpec((BM, D), lambda i: (i, 0)),
            pl.BlockSpec((BM, D), lambda i: (i, 0)),
            pl.BlockSpec((BM, K), lambda i: (i, 0)),
        ],
        out_specs=pl.BlockSpec((BM, D), lambda i: (i, 0)),
        out_shape=jax.ShapeDtypeStruct((S, D), jnp.float32),
    )(a, b, wts)


def kernel(inputs_raw, Wg, w1, w2):
    ishape = inputs_raw.shape
    inputs = inputs_raw.reshape(-1, ishape[-1])

    gate_logits, laux, idx, wts = _router(inputs, Wg)
    l_aux = laux.reshape(())

    e_flat = idx.reshape(-1)
    oh = (e_flat[:, None] == jnp.arange(E)[None, :]).astype(jnp.float32)
    NCH = 32
    CH = A // NCH
    oh2 = oh.reshape(NCH, CH, E).transpose(1, 0, 2).reshape(CH, NCH * E)
    ltri = jnp.tril(jnp.ones((CH, CH), jnp.float32))
    intra2 = ltri @ oh2
    intra = intra2.reshape(CH, NCH, E).transpose(1, 0, 2)
    chunk_tot = intra2[-1].reshape(NCH, E)
    chunk_pre = jnp.cumsum(chunk_tot, axis=0) - chunk_tot
    ranks_incl = intra + chunk_pre[:, None, :]
    ohc = oh.reshape(NCH, CH, E)
    rank = (jnp.sum(ohc * ranks_incl, axis=2).reshape(A) - 1).astype(jnp.int32)
    counts = (chunk_pre[-1] + chunk_tot[-1]).astype(jnp.int32)
    starts = jnp.cumsum(counts) - counts
    ends = starts + counts
    pos = starts[e_flat] + rank
    posT = pos.reshape(NW, TW, K).transpose(0, 2, 1)
    b0 = starts // BM
    b1 = jnp.where(counts > 0, (ends - 1) // BM, b0 - 1)
    nv = b1 - b0 + 1
    cum_nv = jnp.cumsum(nv)
    vstart = cum_nv - nv
    vact = cum_nv[-1]
    vidx = jnp.arange(V)
    ve = jnp.sum((vidx[:, None] >= cum_nv[None, :]).astype(jnp.int32), axis=1)
    ve = jnp.minimum(ve, E - 1)
    last_e = jnp.max(jnp.where(counts > 0, jnp.arange(E), 0))
    valid = vidx < vact
    ve = jnp.where(valid, ve, last_e)
    vb = jnp.where(valid, b0[ve] + (vidx - vstart[ve]), NB - 1)
    vlo = jnp.where(valid, jnp.maximum(starts[ve], vb * BM), 1)
    vhi = jnp.where(valid, jnp.minimum(ends[ve], (vb + 1) * BM), 0)
    meta = jnp.concatenate([ve, vb, vlo, vhi]).astype(jnp.int32)

    x_sorted = _sc_dispatch(inputs, posT)
    y = _expert_mm(meta, x_sorted, w1, w2)
    ya, yb = _sc_uncombine(y, posT)
    results = _combine(ya, yb, wts)

    return (results.reshape(ishape), l_aux, gate_logits)


# speedup vs baseline: 1.0018x; 1.0018x over previous
import jax
from jax import lax
import jax.numpy as jnp
from jax.experimental import pallas as pl
from jax.experimental.pallas import tpu as pltpu
from jax.experimental.pallas import tpu_sc as plsc

S = 2048
D = 1024
F = 2048
E = 8
K = 2
A = S * K
BM = 256
NB = A // BM
V = NB + E - 1

NC = 2
NS = 16
NW = NC * NS
TW = S // NW

def _sc_mesh():
    return plsc.VectorSubcoreMesh(core_axis_name="c", subcore_axis_name="s")


def _dispatch_body(x_hbm, idx_hbm, xs_hbm, ie_v, io_v, rows_v, sem):
    wid = lax.axis_index("s") * NC + lax.axis_index("c")
    pltpu.sync_copy(idx_hbm.at[wid, 0], ie_v)
    pltpu.sync_copy(idx_hbm.at[wid, 1], io_v)
    pltpu.sync_copy(x_hbm.at[pl.ds(wid * TW, TW)], rows_v)
    pltpu.async_copy(rows_v, xs_hbm.at[ie_v], sem).wait()
    pltpu.async_copy(rows_v, xs_hbm.at[io_v], sem).wait()


def _sc_dispatch(inputs, posT):
    kern = pl.kernel(
        _dispatch_body,
        mesh=_sc_mesh(),
        out_type=jax.ShapeDtypeStruct((A, D), jnp.float32),
        scratch_types=[
            pltpu.VMEM((TW,), jnp.int32),
            pltpu.VMEM((TW,), jnp.int32),
            pltpu.VMEM((TW, D), jnp.float32),
            pltpu.SemaphoreType.DMA,
        ],
    )
    return kern(inputs, posT)


def _uncombine_body(y_hbm, idx_hbm, ya_hbm, yb_hbm, ie_v, io_v, rows_v, sem):
    wid = lax.axis_index("s") * NC + lax.axis_index("c")
    pltpu.sync_copy(idx_hbm.at[wid, 0], ie_v)
    pltpu.sync_copy(idx_hbm.at[wid, 1], io_v)
    pltpu.async_copy(y_hbm.at[ie_v], rows_v, sem).wait()
    pltpu.sync_copy(rows_v, ya_hbm.at[pl.ds(wid * TW, TW)])
    pltpu.async_copy(y_hbm.at[io_v], rows_v, sem).wait()
    pltpu.sync_copy(rows_v, yb_hbm.at[pl.ds(wid * TW, TW)])


def _sc_uncombine(y, posT):
    kern = pl.kernel(
        _uncombine_body,
        mesh=_sc_mesh(),
        out_type=[
            jax.ShapeDtypeStruct((S, D), jnp.float32),
            jax.ShapeDtypeStruct((S, D), jnp.float32),
        ],
        scratch_types=[
            pltpu.VMEM((TW,), jnp.int32),
            pltpu.VMEM((TW,), jnp.int32),
            pltpu.VMEM((TW, D), jnp.float32),
            pltpu.SemaphoreType.DMA,
        ],
    )
    return kern(y, posT)


def _router_body(x_ref, wg_ref, gl_ref, laux_ref, idx_ref, wts_ref):
    x = x_ref[...]
    g = jnp.dot(x, wg_ref[...], preferred_element_type=jnp.float32)
    gl_ref[...] = g
    m = jnp.max(g, axis=1, keepdims=True)
    eg = jnp.exp(g - m)
    gates = eg / jnp.sum(eg, axis=1, keepdims=True)
    iota = jax.lax.broadcasted_iota(jnp.int32, (S, E), 1)
    i1 = jnp.min(jnp.where(g == m, iota, E), axis=1)
    sel1 = iota == i1[:, None]
    g2 = jnp.where(sel1, -jnp.inf, g)
    m2 = jnp.max(g2, axis=1, keepdims=True)
    i2 = jnp.min(jnp.where(g2 == m2, iota, E), axis=1)
    w1s = jax.nn.sigmoid(m[:, 0] - m2[:, 0])
    w2s = jax.nn.sigmoid(m2[:, 0] - m[:, 0])
    idx_ref[...] = jnp.concatenate([i1[:, None], i2[:, None]], axis=1)
    wts_ref[...] = jnp.concatenate([w1s[:, None], w2s[:, None]], axis=1)
    me = jnp.mean(gates, axis=0, keepdims=True)
    ce = jnp.mean(sel1.astype(jnp.float32), axis=0, keepdims=True)
    laux_ref[...] = jnp.sum(me * ce, axis=1, keepdims=True) * E


def _router(inputs, Wg):
    return pl.pallas_call(
        _router_body,
        out_shape=[
            jax.ShapeDtypeStruct((S, E), jnp.float32),
            jax.ShapeDtypeStruct((1, 1), jnp.float32),
            jax.ShapeDtypeStruct((S, K), jnp.int32),
            jax.ShapeDtypeStruct((S, K), jnp.float32),
        ],
    )(inputs, Wg)


def _expert_body(meta_ref, x_ref, w1_ref, w2_ref, y_ref):
    v = pl.program_id(0)
    lo = meta_ref[2 * V + v]
    hi = meta_ref[3 * V + v]

    @pl.when(hi > lo)
    def _():
        x = x_ref[...]
        h = jnp.dot(x, w1_ref[0], preferred_element_type=jnp.float32)
        h = h * jax.nn.sigmoid(h)
        y = jnp.dot(h, w2_ref[0], preferred_element_type=jnp.float32)
        r = (meta_ref[V + v] * BM
             + lax.broadcasted_iota(jnp.int32, (BM, 1), 0))
        mask = (r >= lo) & (r < hi)
        y_ref[...] = jnp.where(mask, y, y_ref[...])


def _expert_mm(meta, x_sorted, w1, w2):
    grid_spec = pltpu.PrefetchScalarGridSpec(
        num_scalar_prefetch=1,
        grid=(V,),
        in_specs=[
            pl.BlockSpec((BM, D), lambda v, m: (m[V + v], 0)),
            pl.BlockSpec((1, D, F), lambda v, m: (m[v], 0, 0)),
            pl.BlockSpec((1, F, D), lambda v, m: (m[v], 0, 0)),
        ],
        out_specs=pl.BlockSpec((BM, D), lambda v, m: (m[V + v], 0)),
    )
    return pl.pallas_call(
        _expert_body,
        grid_spec=grid_spec,
        out_shape=jax.ShapeDtypeStruct((A, D), jnp.float32),
        compiler_params=pltpu.CompilerParams(
            dimension_semantics=("arbitrary",)),
    )(meta, x_sorted, w1, w2)


def _combine_body(a_ref, b_ref, w_ref, o_ref):
    w = w_ref[...]
    o_ref[...] = a_ref[...] * w[:, 0:1] + b_ref[...] * w[:, 1:2]


def _combine(a, b, wts):
    nblk = S // BM
    return pl.pallas_call(
        _combine_body,
        grid=(nblk,),
        in_specs=[
            pl.BlockSpec((BM, D), lambda i: (i, 0)),
            pl.BlockSpec((BM, D), lambda i: (i, 0)),
            pl.BlockSpec((BM, K), lambda i: (i, 0)),
        ],
        out_specs=pl.BlockSpec((BM, D), lambda i: (i, 0)),
        out_shape=jax.ShapeDtypeStruct((S, D), jnp.float32),
    )(a, b, wts)


def kernel(inputs_raw, Wg, w1, w2):
    ishape = inputs_raw.shape
    inputs = inputs_raw.reshape(-1, ishape[-1])

    gate_logits, laux, idx, wts = _router(inputs, Wg)
    l_aux = laux.reshape(())

    e_flat = idx.reshape(-1)
    oh = (e_flat[:, None] == jnp.arange(E)[None, :]).astype(jnp.float32)
    NCH = 32
    CH = A // NCH
    oh2 = oh.reshape(NCH, CH, E).transpose(1, 0, 2).reshape(CH, NCH * E)
    ltri = jnp.tril(jnp.ones((CH, CH), jnp.float32))
    intra2 = ltri @ oh2
    intra = intra2.reshape(CH, NCH, E).transpose(1, 0, 2)
    chunk_tot = intra2[-1].reshape(NCH, E)
    chunk_pre = jnp.cumsum(chunk_tot, axis=0) - chunk_tot
    ranks_incl = intra + chunk_pre[:, None, :]
    ohc = oh.reshape(NCH, CH, E)
    rank = (jnp.sum(ohc * ranks_incl, axis=2).reshape(A) - 1).astype(jnp.int32)
    counts = (chunk_pre[-1] + chunk_tot[-1]).astype(jnp.int32)
    starts = jnp.cumsum(counts) - counts
    ends = starts + counts
    pos = starts[e_flat] + rank
    posT = pos.reshape(NW, TW, K).transpose(0, 2, 1)
    b0 = starts // BM
    b1 = jnp.where(counts > 0, (ends - 1) // BM, b0 - 1)
    nv = b1 - b0 + 1
    cum_nv = jnp.cumsum(nv)
    vstart = cum_nv - nv
    vact = cum_nv[-1]
    vidx = jnp.arange(V)
    ve = jnp.sum((vidx[:, None] >= cum_nv[None, :]).astype(jnp.int32), axis=1)
    ve = jnp.minimum(ve, E - 1)
    last_e = jnp.max(jnp.where(counts > 0, jnp.arange(E), 0))
    valid = vidx < vact
    ve = jnp.where(valid, ve, last_e)
    vb = jnp.where(valid, b0[ve] + (vidx - vstart[ve]), NB - 1)
    vlo = jnp.where(valid, jnp.maximum(starts[ve], vb * BM), 1)
    vhi = jnp.where(valid, jnp.minimum(ends[ve], (vb + 1) * BM), 0)
    meta = jnp.concatenate([ve, vb, vlo, vhi]).astype(jnp.int32)

    x_sorted = _sc_dispatch(inputs, posT)
    y = _expert_mm(meta, x_sorted, w1, w2)
    ya, yb = _sc_uncombine(y, posT)
    results = _combine(ya, yb, wts)

    return (results.reshape(ishape), l_aux, gate_logits)


# speedup vs baseline: 1.0842x; 1.0823x over previous
import jax
from jax import lax
import jax.numpy as jnp
from jax.experimental import pallas as pl
from jax.experimental.pallas import tpu as pltpu
from jax.experimental.pallas import tpu_sc as plsc

S = 2048
D = 1024
F = 2048
E = 8
K = 2
A = S * K
BM = 256
NB = A // BM
V = NB + E - 1

NC = 2
NS = 16
NW = NC * NS
TW = S // NW

def _sc_mesh():
    return plsc.VectorSubcoreMesh(core_axis_name="c", subcore_axis_name="s")


def _dispatch_body(x_hbm, idx_hbm, xs_hbm, ie_v, io_v, rows_v, sem):
    wid = lax.axis_index("s") * NC + lax.axis_index("c")
    pltpu.sync_copy(idx_hbm.at[wid, 0], ie_v)
    pltpu.sync_copy(idx_hbm.at[wid, 1], io_v)
    pltpu.sync_copy(x_hbm.at[pl.ds(wid * TW, TW)], rows_v)
    pltpu.async_copy(rows_v, xs_hbm.at[ie_v], sem).wait()
    pltpu.async_copy(rows_v, xs_hbm.at[io_v], sem).wait()


def _sc_dispatch(inputs, posT):
    kern = pl.kernel(
        _dispatch_body,
        mesh=_sc_mesh(),
        out_type=jax.ShapeDtypeStruct((A, D), jnp.float32),
        scratch_types=[
            pltpu.VMEM((TW,), jnp.int32),
            pltpu.VMEM((TW,), jnp.int32),
            pltpu.VMEM((TW, D), jnp.float32),
            pltpu.SemaphoreType.DMA,
        ],
    )
    return kern(inputs, posT)


def _uncombine_body(y_hbm, idx_hbm, ya_hbm, yb_hbm, ie_v, io_v, rows_v, sem):
    wid = lax.axis_index("s") * NC + lax.axis_index("c")
    pltpu.sync_copy(idx_hbm.at[wid, 0], ie_v)
    pltpu.sync_copy(idx_hbm.at[wid, 1], io_v)
    pltpu.async_copy(y_hbm.at[ie_v], rows_v, sem).wait()
    pltpu.sync_copy(rows_v, ya_hbm.at[pl.ds(wid * TW, TW)])
    pltpu.async_copy(y_hbm.at[io_v], rows_v, sem).wait()
    pltpu.sync_copy(rows_v, yb_hbm.at[pl.ds(wid * TW, TW)])


def _sc_uncombine(y, posT):
    kern = pl.kernel(
        _uncombine_body,
        mesh=_sc_mesh(),
        out_type=[
            jax.ShapeDtypeStruct((S, D), jnp.float32),
            jax.ShapeDtypeStruct((S, D), jnp.float32),
        ],
        scratch_types=[
            pltpu.VMEM((TW,), jnp.int32),
            pltpu.VMEM((TW,), jnp.int32),
            pltpu.VMEM((TW, D), jnp.float32),
            pltpu.SemaphoreType.DMA,
        ],
    )
    return kern(y, posT)


NCH = 32
CH = A // NCH


def _router_body(x_ref, wg_ref, gl_ref, laux_ref, wts_ref, pos_ref, meta_ref):
    x = x_ref[...]
    g = jnp.dot(x, wg_ref[...], preferred_element_type=jnp.float32)
    gl_ref[...] = g
    m = jnp.max(g, axis=1, keepdims=True)
    eg = jnp.exp(g - m)
    gates = eg / jnp.sum(eg, axis=1, keepdims=True)
    iota = jax.lax.broadcasted_iota(jnp.int32, (S, E), 1)
    i1 = jnp.min(jnp.where(g == m, iota, E), axis=1)
    sel1 = iota == i1[:, None]
    g2 = jnp.where(sel1, -jnp.inf, g)
    m2 = jnp.max(g2, axis=1, keepdims=True)
    sel2 = (iota == jnp.min(jnp.where(g2 == m2, iota, E), axis=1)[:, None])
    w1s = jax.nn.sigmoid(m[:, 0] - m2[:, 0])
    w2s = jax.nn.sigmoid(m2[:, 0] - m[:, 0])
    wts_ref[...] = jnp.concatenate([w1s[:, None], w2s[:, None]], axis=1)
    me = jnp.mean(gates, axis=0, keepdims=True)
    ce = jnp.mean(sel1.astype(jnp.float32), axis=0, keepdims=True)
    laux_ref[...] = jnp.sum(me * ce, axis=1, keepdims=True) * E

    ecat = jnp.concatenate([sel1.astype(jnp.float32),
                            sel2.astype(jnp.float32)], axis=0)
    ohc = ecat.reshape(NCH, CH, E)
    r2 = lax.broadcasted_iota(jnp.int32, (CH, CH), 0)
    c2 = lax.broadcasted_iota(jnp.int32, (CH, CH), 1)
    ltri = jnp.where(r2 >= c2, 1.0, 0.0)
    ltri_b = jnp.broadcast_to(ltri[None], (NCH, CH, CH))
    intra = lax.dot_general(ltri_b, ohc, (((2,), (1,)), ((0,), (0,))))
    chunk_tot = intra[:, CH - 1, :]
    rn = lax.broadcasted_iota(jnp.int32, (NCH, NCH), 0)
    cn = lax.broadcasted_iota(jnp.int32, (NCH, NCH), 1)
    sltri = jnp.where(rn > cn, 1.0, 0.0)
    chunk_pre = lax.dot_general(sltri, chunk_tot,
                                (((1,), (0,)), ((), ())))
    ranks_incl = intra + chunk_pre[:, None, :]
    counts = (chunk_pre[NCH - 1] + chunk_tot[NCH - 1])[None]
    r8 = lax.broadcasted_iota(jnp.int32, (E, E), 0)
    c8 = lax.broadcasted_iota(jnp.int32, (E, E), 1)
    sue = jnp.where(r8 < c8, 1.0, 0.0)
    starts = lax.dot_general(counts, sue, (((1,), (0,)), ((), ())))
    posf = jnp.sum(ohc * (ranks_incl - 1.0 + starts[None]), axis=2)
    pos_ref[...] = posf.astype(jnp.int32)

    countsi = counts.astype(jnp.int32)
    startsi = starts.astype(jnp.int32)
    endsi = startsi + countsi
    b0 = startsi // BM
    b1 = jnp.where(countsi > 0, (endsi - 1) // BM, b0 - 1)
    nv = b1 - b0 + 1
    cum_nvf = lax.dot_general(nv.astype(jnp.float32),
                              jnp.where(r8 <= c8, 1.0, 0.0),
                              (((1,), (0,)), ((), ())))
    cum_nv = cum_nvf.astype(jnp.int32)
    vstart = cum_nv - nv
    vact = cum_nv[0, E - 1]
    vidx = lax.broadcasted_iota(jnp.int32, (V, 1), 0)
    iota8r = lax.broadcasted_iota(jnp.int32, (V, E), 1)
    ve = jnp.sum((vidx >= cum_nv).astype(jnp.int32), axis=1,
                 keepdims=True)
    ve = jnp.minimum(ve, E - 1)
    e_live = jnp.where(countsi > 0, iota8r[:1], 0)
    last_e = jnp.max(e_live, axis=1, keepdims=True)
    valid = vidx < vact
    ve = jnp.where(valid, ve, last_e)
    oh_ve = (iota8r == ve).astype(jnp.int32)
    b0_v = jnp.sum(oh_ve * b0, axis=1, keepdims=True)
    vs_v = jnp.sum(oh_ve * vstart, axis=1, keepdims=True)
    st_v = jnp.sum(oh_ve * startsi, axis=1, keepdims=True)
    en_v = jnp.sum(oh_ve * endsi, axis=1, keepdims=True)
    vb = jnp.where(valid, b0_v + (vidx - vs_v), NB - 1)
    vlo = jnp.where(valid, jnp.maximum(st_v, vb * BM), 1)
    vhi = jnp.where(valid, jnp.minimum(en_v, (vb + 1) * BM), 0)
    meta_ref[...] = jnp.concatenate(
        [ve.T, vb.T, vlo.T, vhi.T], axis=0)


def _router(inputs, Wg):
    return pl.pallas_call(
        _router_body,
        out_shape=[
            jax.ShapeDtypeStruct((S, E), jnp.float32),
            jax.ShapeDtypeStruct((1, 1), jnp.float32),
            jax.ShapeDtypeStruct((S, K), jnp.float32),
            jax.ShapeDtypeStruct((NCH, CH), jnp.int32),
            jax.ShapeDtypeStruct((4, V), jnp.int32),
        ],
    )(inputs, Wg)


def _expert_body(meta_ref, x_ref, w1_ref, w2_ref, y_ref):
    v = pl.program_id(0)
    lo = meta_ref[2 * V + v]
    hi = meta_ref[3 * V + v]

    @pl.when(hi > lo)
    def _():
        x = x_ref[...]
        h = jnp.dot(x, w1_ref[0], preferred_element_type=jnp.float32)
        h = h * jax.nn.sigmoid(h)
        y = jnp.dot(h, w2_ref[0], preferred_element_type=jnp.float32)
        r = (meta_ref[V + v] * BM
             + lax.broadcasted_iota(jnp.int32, (BM, 1), 0))
        mask = (r >= lo) & (r < hi)
        y_ref[...] = jnp.where(mask, y, y_ref[...])


def _expert_mm(meta, x_sorted, w1, w2):
    grid_spec = pltpu.PrefetchScalarGridSpec(
        num_scalar_prefetch=1,
        grid=(V,),
        in_specs=[
            pl.BlockSpec((BM, D), lambda v, m: (m[V + v], 0)),
            pl.BlockSpec((1, D, F), lambda v, m: (m[v], 0, 0)),
            pl.BlockSpec((1, F, D), lambda v, m: (m[v], 0, 0)),
        ],
        out_specs=pl.BlockSpec((BM, D), lambda v, m: (m[V + v], 0)),
    )
    return pl.pallas_call(
        _expert_body,
        grid_spec=grid_spec,
        out_shape=jax.ShapeDtypeStruct((A, D), jnp.float32),
        compiler_params=pltpu.CompilerParams(
            dimension_semantics=("arbitrary",)),
    )(meta, x_sorted, w1, w2)


def _combine_body(a_ref, b_ref, w_ref, o_ref):
    w = w_ref[...]
    o_ref[...] = a_ref[...] * w[:, 0:1] + b_ref[...] * w[:, 1:2]


def _combine(a, b, wts):
    nblk = S // BM
    return pl.pallas_call(
        _combine_body,
        grid=(nblk,),
        in_specs=[
            pl.BlockSpec((BM, D), lambda i: (i, 0)),
            pl.BlockSpec((BM, D), lambda i: (i, 0)),
            pl.BlockSpec((BM, K), lambda i: (i, 0)),
        ],
        out_specs=pl.BlockSpec((BM, D), lambda i: (i, 0)),
        out_shape=jax.ShapeDtypeStruct((S, D), jnp.float32),
    )(a, b, wts)


def kernel(inputs_raw, Wg, w1, w2):
    ishape = inputs_raw.shape
    inputs = inputs_raw.reshape(-1, ishape[-1])

    gate_logits, laux, wts, pos_out, meta_out = _router(inputs, Wg)
    l_aux = laux.reshape(())
    posT = pos_out.reshape(K, NW, TW).transpose(1, 0, 2)
    meta = meta_out.reshape(4 * V)

    x_sorted = _sc_dispatch(inputs, posT)
    y = _expert_mm(meta, x_sorted, w1, w2)
    ya, yb = _sc_uncombine(y, posT)
    results = _combine(ya, yb, wts)

    return (results.reshape(ishape), l_aux, gate_logits)
